# Initial kernel scaffold; baseline (speedup 1.0000x reference)
import jax
import jax.numpy as jnp
from jax.experimental import pallas as pl


def kernel(x, edge_index, batch, W1, b1, W2, b2, W3, b3, Wl, bl):
    raise NotImplementedError("write your pallas kernel here")


# speedup vs baseline: 4.2537x; 4.2537x over previous
import functools

import jax
import jax.numpy as jnp
from jax import lax
from jax.experimental import pallas as pl
from jax.experimental.pallas import tpu as pltpu
from jax.experimental.pallas import tpu_sc as plsc

N = 10000
NP = 10240
E = 320000
D = 128
G = 64
C = 16
CH = 128
NW = 32
RPW = NP // NW
PADN = N + 100
TCB = 1024
NT = NP // TCB
NSEG = D // 16


def _sc_mesh():
    return plsc.VectorSubcoreMesh(core_axis_name="c", subcore_axis_name="s")


@functools.partial(
    pl.kernel,
    out_type=jax.ShapeDtypeStruct((NP, 16), jnp.float32),
    mesh=_sc_mesh(),
    scratch_types=[
        pltpu.VMEM((48,), jnp.int32),
        pltpu.VMEM((CH + 16,), jnp.int32),
        pltpu.VMEM((RPW, 16), jnp.float32),
    ],
)
def _sc_deg(dst_hbm, st_hbm, out_hbm, st_v, idx_d, acc):
    c = lax.axis_index("c")
    s = lax.axis_index("s")
    w = c * 16 + s
    base = w * RPW

    def _z(i, _):
        acc[i] = jnp.zeros((16,), jnp.float32)
        return 0

    lax.fori_loop(0, RPW, _z, 0)

    def _zi(j, _):
        idx_d[pl.ds(j * 16, 16)] = jnp.zeros((16,), jnp.int32)
        return 0

    lax.fori_loop(0, (CH + 16) // 16, _zi, 0)

    def _zs(j, _):
        st_v[pl.ds(j * 16, 16)] = jnp.zeros((16,), jnp.int32)
        return 0

    lax.fori_loop(0, 3, _zs, 0)
    pltpu.sync_copy(st_hbm, st_v.at[pl.ds(0, 40)])
    e0 = st_v[pl.ds(w, 16)][0]
    e1 = st_v[pl.ds(w + 1, 16)][0]
    a0 = (e0 // CH) * CH
    ones = jnp.ones((16,), jnp.float32)

    def chunk(k, _):
        start = a0 + k * CH
        lo = jnp.maximum(e0 - start, 0)
        hi = jnp.maximum(jnp.minimum(e1 - start, CH), lo)
        pltpu.sync_copy(dst_hbm.at[pl.ds(start, CH)], idx_d.at[pl.ds(0, CH)])

        def edge(e, _):
            r = idx_d[pl.ds(e, 16)][0] - base
            acc[r] = acc[r] + ones
            return 0

        lax.fori_loop(lo, hi, edge, 0)
        return 0

    lax.fori_loop(0, jnp.maximum((e1 - a0 + CH - 1) // CH, 0), chunk, 0)
    pltpu.sync_copy(acc, out_hbm.at[pl.ds(base, RPW)])


@functools.partial(
    pl.kernel,
    out_type=jax.ShapeDtypeStruct((NP, D), jnp.float32),
    mesh=_sc_mesh(),
    scratch_types=[
        pltpu.VMEM((48,), jnp.int32),
        pltpu.VMEM((CH,), jnp.int32),
        pltpu.VMEM((CH + 16,), jnp.int32),
        pltpu.VMEM((CH, D), jnp.float32),
        pltpu.VMEM((RPW, D), jnp.float32),
        pltpu.SemaphoreType.DMA,
    ],
)
def _sc_agg(h_hbm, src_hbm, dst_hbm, st_hbm, out_hbm, st_v, idx_s, idx_d,
            rows, acc, sem):
    c = lax.axis_index("c")
    s = lax.axis_index("s")
    w = c * 16 + s
    base = w * RPW

    def _z(i, _):
        for j in range(NSEG):
            acc[i, pl.ds(j * 16, 16)] = jnp.zeros((16,), jnp.float32)
        return 0

    lax.fori_loop(0, RPW, _z, 0)

    def _zi(j, _):
        idx_s[pl.ds(j * 16, 16)] = jnp.zeros((16,), jnp.int32)
        idx_d[pl.ds(j * 16, 16)] = jnp.zeros((16,), jnp.int32)
        return 0

    lax.fori_loop(0, CH // 16, _zi, 0)
    idx_d[pl.ds(CH, 16)] = jnp.zeros((16,), jnp.int32)

    def _zs(j, _):
        st_v[pl.ds(j * 16, 16)] = jnp.zeros((16,), jnp.int32)
        return 0

    lax.fori_loop(0, 3, _zs, 0)
    pltpu.sync_copy(st_hbm, st_v.at[pl.ds(0, 40)])
    e0 = st_v[pl.ds(w, 16)][0]
    e1 = st_v[pl.ds(w + 1, 16)][0]
    a0 = (e0 // CH) * CH

    def chunk(k, carry):
        start = a0 + k * CH
        lo = jnp.maximum(e0 - start, 0)
        hi = jnp.maximum(jnp.minimum(e1 - start, CH), lo)
        pltpu.sync_copy(src_hbm.at[pl.ds(start, CH)], idx_s)
        pltpu.sync_copy(dst_hbm.at[pl.ds(start, CH)], idx_d.at[pl.ds(0, CH)])
        pltpu.async_copy(h_hbm.at[idx_s], rows, sem).wait()

        def edge(e, car):
            r = idx_d[pl.ds(e, 16)][0] - base
            for j in range(NSEG):
                acc[r, pl.ds(j * 16, 16)] = (
                    acc[r, pl.ds(j * 16, 16)] + rows[e, pl.ds(j * 16, 16)]
                )
            return car

        return lax.fori_loop(lo, hi, edge, carry)

    lax.fori_loop(0, jnp.maximum((e1 - a0 + CH - 1) // CH, 0), chunk, 0)
    pltpu.sync_copy(acc, out_hbm.at[pl.ds(base, RPW)])


def _tc_prep_body(degp_ref, x_ref, dis_ref, h0_ref):
    t = pl.program_id(0)
    d = degp_ref[:, 0:1] + 1.0
    gid = t * TCB + lax.broadcasted_iota(jnp.int32, (TCB, 1), 0)
    dis = jnp.where(gid < N, lax.rsqrt(d), 0.0)
    dis_ref[...] = dis
    h0_ref[...] = x_ref[...] * dis


def _tc_prep(degp, x_p):
    return pl.pallas_call(
        _tc_prep_body,
        grid=(NT,),
        in_specs=[
            pl.BlockSpec((TCB, 16), lambda t: (t, 0)),
            pl.BlockSpec((TCB, D), lambda t: (t, 0)),
        ],
        out_specs=[
            pl.BlockSpec((TCB, 1), lambda t: (t, 0)),
            pl.BlockSpec((TCB, D), lambda t: (t, 0)),
        ],
        out_shape=[
            jax.ShapeDtypeStruct((NP, 1), jnp.float32),
            jax.ShapeDtypeStruct((NP, D), jnp.float32),
        ],
    )(degp, x_p)


def _tc_layer_body(acc_ref, hp_ref, dis_ref, w_ref, b_ref, out_ref):
    dis = dis_ref[...]
    u = (acc_ref[...] + hp_ref[...]) * dis
    z = jnp.dot(u, w_ref[...], preferred_element_type=jnp.float32) + b_ref[...]
    out_ref[...] = jnp.maximum(z, 0.0) * dis


def _tc_layer(agg, hp, dis, W, b):
    return pl.pallas_call(
        _tc_layer_body,
        grid=(NT,),
        in_specs=[
            pl.BlockSpec((TCB, D), lambda t: (t, 0)),
            pl.BlockSpec((TCB, D), lambda t: (t, 0)),
            pl.BlockSpec((TCB, 1), lambda t: (t, 0)),
            pl.BlockSpec((D, D), lambda t: (0, 0)),
            pl.BlockSpec((1, D), lambda t: (0, 0)),
        ],
        out_specs=pl.BlockSpec((TCB, D), lambda t: (t, 0)),
        out_shape=jax.ShapeDtypeStruct((NP, D), jnp.float32),
    )(agg, hp, dis, W, b)


def _tc_final_body(acc_ref, hp_ref, dis_ref, w_ref, b_ref, bat_ref, wl_ref,
                   bl_ref, out_ref, sums, cnts):
    t = pl.program_id(0)
    u = (acc_ref[...] + hp_ref[...]) * dis_ref[...]
    h3 = jnp.dot(u, w_ref[...], preferred_element_type=jnp.float32) + b_ref[...]
    gids = lax.broadcasted_iota(jnp.int32, (TCB, G), 1)
    P = (bat_ref[...] == gids).astype(jnp.float32)
    psum = lax.dot_general(P, h3, (((0,), (0,)), ((), ())),
                           preferred_element_type=jnp.float32)
    pcnt = lax.dot_general(P, jnp.ones((TCB, D), jnp.float32),
                           (((0,), (0,)), ((), ())),
                           preferred_element_type=jnp.float32)

    @pl.when(t == 0)
    def _():
        sums[...] = psum
        cnts[...] = pcnt

    @pl.when(t > 0)
    def _():
        sums[...] += psum
        cnts[...] += pcnt

    @pl.when(t == NT - 1)
    def _():
        means = sums[...] / jnp.maximum(cnts[...], 1.0)
        out_ref[...] = jnp.dot(means, wl_ref[...],
                               preferred_element_type=jnp.float32) + bl_ref[...]


def _tc_final(agg, hp, dis, W, b, bat, Wl, bl):
    return pl.pallas_call(
        _tc_final_body,
        grid=(NT,),
        in_specs=[
            pl.BlockSpec((TCB, D), lambda t: (t, 0)),
            pl.BlockSpec((TCB, D), lambda t: (t, 0)),
            pl.BlockSpec((TCB, 1), lambda t: (t, 0)),
            pl.BlockSpec((D, D), lambda t: (0, 0)),
            pl.BlockSpec((1, D), lambda t: (0, 0)),
            pl.BlockSpec((TCB, 1), lambda t: (t, 0)),
            pl.BlockSpec((D, C), lambda t: (0, 0)),
            pl.BlockSpec((1, C), lambda t: (0, 0)),
        ],
        out_specs=pl.BlockSpec((G, C), lambda t: (0, 0)),
        out_shape=jax.ShapeDtypeStruct((G, C), jnp.float32),
        scratch_shapes=[
            pltpu.VMEM((G, D), jnp.float32),
            pltpu.VMEM((G, D), jnp.float32),
        ],
    )(agg, hp, dis, W, b, bat, Wl, bl)


def kernel(x, edge_index, batch, W1, b1, W2, b2, W3, b3, Wl, bl):
    src = edge_index[0].astype(jnp.int32)
    dst = edge_index[1].astype(jnp.int32)
    order = jnp.argsort(dst)
    dsts = dst[order]
    srcs = src[order]
    srcs_p = jnp.concatenate([srcs, jnp.full((CH,), PADN, jnp.int32)])
    dsts_p = jnp.concatenate([dsts, jnp.full((CH,), NP - 1, jnp.int32)])
    starts = jnp.searchsorted(
        dsts, jnp.arange(NW + 1, dtype=jnp.int32) * RPW
    ).astype(jnp.int32)
    starts = jnp.concatenate([starts, jnp.zeros((7,), jnp.int32)])
    x_p = jnp.concatenate([x, jnp.zeros((NP - N, D), x.dtype)])
    bat_p = jnp.concatenate(
        [batch.astype(jnp.int32), jnp.full((NP - N,), G, jnp.int32)]
    ).reshape(NP, 1)

    degp = _sc_deg(dsts_p, starts)
    dis, h0 = _tc_prep(degp, x_p)
    agg1 = _sc_agg(h0, srcs_p, dsts_p, starts)
    h1 = _tc_layer(agg1, h0, dis, W1, b1.reshape(1, D))
    agg2 = _sc_agg(h1, srcs_p, dsts_p, starts)
    h2 = _tc_layer(agg2, h1, dis, W2, b2.reshape(1, D))
    agg3 = _sc_agg(h2, srcs_p, dsts_p, starts)
    return _tc_final(agg3, h2, dis, W3, b3.reshape(1, D), bat_p, Wl,
                     bl.reshape(1, C))


# speedup vs baseline: 5.1516x; 1.2111x over previous
import functools

import jax
import jax.numpy as jnp
from jax import lax
from jax.experimental import pallas as pl
from jax.experimental.pallas import tpu as pltpu
from jax.experimental.pallas import tpu_sc as plsc

N = 10000
NP = 10240
E = 320000
D = 128
G = 64
C = 16
CH = 128
NW = 32
RPW = NP // NW
PADN = N + 100
TCB = 1024
NT = NP // TCB
NSEG = D // 16


def _sc_mesh():
    return plsc.VectorSubcoreMesh(core_axis_name="c", subcore_axis_name="s")


@functools.partial(
    pl.kernel,
    out_type=jax.ShapeDtypeStruct((NP, 16), jnp.float32),
    mesh=_sc_mesh(),
    scratch_types=[
        pltpu.VMEM((48,), jnp.int32),
        pltpu.VMEM((CH + 16,), jnp.int32),
        pltpu.VMEM((RPW + 8, 16), jnp.float32),
    ],
)
def _sc_deg(dst_hbm, st_hbm, out_hbm, st_v, idx_d, acc):
    c = lax.axis_index("c")
    s = lax.axis_index("s")
    w = c * 16 + s
    base = w * RPW

    def _z(i, _):
        acc[i] = jnp.zeros((16,), jnp.float32)
        return 0

    lax.fori_loop(0, RPW + 8, _z, 0)

    def _zi(j, _):
        idx_d[pl.ds(j * 16, 16)] = jnp.zeros((16,), jnp.int32)
        return 0

    lax.fori_loop(0, (CH + 16) // 16, _zi, 0)

    def _zs(j, _):
        st_v[pl.ds(j * 16, 16)] = jnp.zeros((16,), jnp.int32)
        return 0

    lax.fori_loop(0, 3, _zs, 0)
    pltpu.sync_copy(st_hbm, st_v.at[pl.ds(0, 40)])
    e0 = st_v[pl.ds(w, 16)][0]
    e1 = st_v[pl.ds(w + 1, 16)][0]
    a0 = (e0 // CH) * CH
    ones = jnp.ones((16,), jnp.float32)

    def chunk(k, _):
        start = a0 + k * CH
        lo = e0 - start
        hi = jnp.minimum(e1 - start, CH)
        pltpu.sync_copy(dst_hbm.at[pl.ds(start, CH)], idx_d.at[pl.ds(0, CH)])

        def group(g, _):
            dvec = idx_d[pl.ds(g * 16, 16)]
            gbase = g * 16
            for i in range(16):
                e = gbase + i
                inb = jnp.logical_and(e >= lo, e < hi)
                r = jnp.where(inb, dvec[i] - base, jnp.int32(RPW))
                acc[r] = acc[r] + ones
            return 0

        lax.fori_loop(0, CH // 16, group, 0)
        return 0

    lax.fori_loop(0, jnp.maximum((e1 - a0 + CH - 1) // CH, 0), chunk, 0)
    pltpu.sync_copy(acc.at[pl.ds(0, RPW)], out_hbm.at[pl.ds(base, RPW)])


@functools.partial(
    pl.kernel,
    out_type=jax.ShapeDtypeStruct((NP, D), jnp.float32),
    mesh=_sc_mesh(),
    scratch_types=[
        pltpu.VMEM((48,), jnp.int32),
        pltpu.VMEM((CH,), jnp.int32),
        pltpu.VMEM((CH + 16,), jnp.int32),
        pltpu.VMEM((CH, D), jnp.float32),
        pltpu.VMEM((RPW + 8, D), jnp.float32),
        pltpu.SemaphoreType.DMA,
    ],
)
def _sc_agg(h_hbm, src_hbm, dst_hbm, st_hbm, out_hbm, st_v, idx_s, idx_d,
            rows, acc, sem):
    c = lax.axis_index("c")
    s = lax.axis_index("s")
    w = c * 16 + s
    base = w * RPW

    def _z(i, _):
        for j in range(NSEG):
            acc[i, pl.ds(j * 16, 16)] = jnp.zeros((16,), jnp.float32)
        return 0

    lax.fori_loop(0, RPW + 8, _z, 0)

    def _zi(j, _):
        idx_s[pl.ds(j * 16, 16)] = jnp.zeros((16,), jnp.int32)
        idx_d[pl.ds(j * 16, 16)] = jnp.zeros((16,), jnp.int32)
        return 0

    lax.fori_loop(0, CH // 16, _zi, 0)
    idx_d[pl.ds(CH, 16)] = jnp.zeros((16,), jnp.int32)

    def _zs(j, _):
        st_v[pl.ds(j * 16, 16)] = jnp.zeros((16,), jnp.int32)
        return 0

    lax.fori_loop(0, 3, _zs, 0)
    pltpu.sync_copy(st_hbm, st_v.at[pl.ds(0, 40)])
    e0 = st_v[pl.ds(w, 16)][0]
    e1 = st_v[pl.ds(w + 1, 16)][0]
    a0 = (e0 // CH) * CH

    def chunk(k, carry):
        start = a0 + k * CH
        lo = e0 - start
        hi = jnp.minimum(e1 - start, CH)
        pltpu.sync_copy(src_hbm.at[pl.ds(start, CH)], idx_s)
        pltpu.sync_copy(dst_hbm.at[pl.ds(start, CH)], idx_d.at[pl.ds(0, CH)])
        pltpu.async_copy(h_hbm.at[idx_s], rows, sem).wait()

        def group(g, car):
            dvec = idx_d[pl.ds(g * 16, 16)]
            gbase = g * 16
            for i in range(16):
                e = gbase + i
                inb = jnp.logical_and(e >= lo, e < hi)
                r = jnp.where(inb, dvec[i] - base, jnp.int32(RPW))
                for j in range(NSEG):
                    acc[r, pl.ds(j * 16, 16)] = (
                        acc[r, pl.ds(j * 16, 16)] + rows[e, pl.ds(j * 16, 16)]
                    )
            return car

        return lax.fori_loop(0, CH // 16, group, carry)

    lax.fori_loop(0, jnp.maximum((e1 - a0 + CH - 1) // CH, 0), chunk, 0)
    pltpu.sync_copy(acc.at[pl.ds(0, RPW)], out_hbm.at[pl.ds(base, RPW)])


def _tc_prep_body(degp_ref, x_ref, dis_ref, h0_ref):
    t = pl.program_id(0)
    d = degp_ref[:, 0:1] + 1.0
    gid = t * TCB + lax.broadcasted_iota(jnp.int32, (TCB, 1), 0)
    dis = jnp.where(gid < N, lax.rsqrt(d), 0.0)
    dis_ref[...] = dis
    h0_ref[...] = x_ref[...] * dis


def _tc_prep(degp, x_p):
    return pl.pallas_call(
        _tc_prep_body,
        grid=(NT,),
        in_specs=[
            pl.BlockSpec((TCB, 16), lambda t: (t, 0)),
            pl.BlockSpec((TCB, D), lambda t: (t, 0)),
        ],
        out_specs=[
            pl.BlockSpec((TCB, 1), lambda t: (t, 0)),
            pl.BlockSpec((TCB, D), lambda t: (t, 0)),
        ],
        out_shape=[
            jax.ShapeDtypeStruct((NP, 1), jnp.float32),
            jax.ShapeDtypeStruct((NP, D), jnp.float32),
        ],
    )(degp, x_p)


def _tc_layer_body(acc_ref, hp_ref, dis_ref, w_ref, b_ref, out_ref):
    dis = dis_ref[...]
    u = (acc_ref[...] + hp_ref[...]) * dis
    z = jnp.dot(u, w_ref[...], preferred_element_type=jnp.float32) + b_ref[...]
    out_ref[...] = jnp.maximum(z, 0.0) * dis


def _tc_layer(agg, hp, dis, W, b):
    return pl.pallas_call(
        _tc_layer_body,
        grid=(NT,),
        in_specs=[
            pl.BlockSpec((TCB, D), lambda t: (t, 0)),
            pl.BlockSpec((TCB, D), lambda t: (t, 0)),
            pl.BlockSpec((TCB, 1), lambda t: (t, 0)),
            pl.BlockSpec((D, D), lambda t: (0, 0)),
            pl.BlockSpec((1, D), lambda t: (0, 0)),
        ],
        out_specs=pl.BlockSpec((TCB, D), lambda t: (t, 0)),
        out_shape=jax.ShapeDtypeStruct((NP, D), jnp.float32),
    )(agg, hp, dis, W, b)


def _tc_final_body(acc_ref, hp_ref, dis_ref, w_ref, b_ref, bat_ref, wl_ref,
                   bl_ref, out_ref, sums, cnts):
    t = pl.program_id(0)
    u = (acc_ref[...] + hp_ref[...]) * dis_ref[...]
    h3 = jnp.dot(u, w_ref[...], preferred_element_type=jnp.float32) + b_ref[...]
    gids = lax.broadcasted_iota(jnp.int32, (TCB, G), 1)
    P = (bat_ref[...] == gids).astype(jnp.float32)
    psum = lax.dot_general(P, h3, (((0,), (0,)), ((), ())),
                           preferred_element_type=jnp.float32)
    pcnt = lax.dot_general(P, jnp.ones((TCB, D), jnp.float32),
                           (((0,), (0,)), ((), ())),
                           preferred_element_type=jnp.float32)

    @pl.when(t == 0)
    def _():
        sums[...] = psum
        cnts[...] = pcnt

    @pl.when(t > 0)
    def _():
        sums[...] += psum
        cnts[...] += pcnt

    @pl.when(t == NT - 1)
    def _():
        means = sums[...] / jnp.maximum(cnts[...], 1.0)
        out_ref[...] = jnp.dot(means, wl_ref[...],
                               preferred_element_type=jnp.float32) + bl_ref[...]


def _tc_final(agg, hp, dis, W, b, bat, Wl, bl):
    return pl.pallas_call(
        _tc_final_body,
        grid=(NT,),
        in_specs=[
            pl.BlockSpec((TCB, D), lambda t: (t, 0)),
            pl.BlockSpec((TCB, D), lambda t: (t, 0)),
            pl.BlockSpec((TCB, 1), lambda t: (t, 0)),
            pl.BlockSpec((D, D), lambda t: (0, 0)),
            pl.BlockSpec((1, D), lambda t: (0, 0)),
            pl.BlockSpec((TCB, 1), lambda t: (t, 0)),
            pl.BlockSpec((D, C), lambda t: (0, 0)),
            pl.BlockSpec((1, C), lambda t: (0, 0)),
        ],
        out_specs=pl.BlockSpec((G, C), lambda t: (0, 0)),
        out_shape=jax.ShapeDtypeStruct((G, C), jnp.float32),
        scratch_shapes=[
            pltpu.VMEM((G, D), jnp.float32),
            pltpu.VMEM((G, D), jnp.float32),
        ],
    )(agg, hp, dis, W, b, bat, Wl, bl)


def kernel(x, edge_index, batch, W1, b1, W2, b2, W3, b3, Wl, bl):
    src = edge_index[0].astype(jnp.int32)
    dst = edge_index[1].astype(jnp.int32)
    order = jnp.argsort(dst)
    dsts = dst[order]
    srcs = src[order]
    srcs_p = jnp.concatenate([srcs, jnp.full((CH,), PADN, jnp.int32)])
    dsts_p = jnp.concatenate([dsts, jnp.full((CH,), NP - 1, jnp.int32)])
    starts = jnp.searchsorted(
        dsts, jnp.arange(NW + 1, dtype=jnp.int32) * RPW
    ).astype(jnp.int32)
    starts = jnp.concatenate([starts, jnp.zeros((7,), jnp.int32)])
    x_p = jnp.concatenate([x, jnp.zeros((NP - N, D), x.dtype)])
    bat_p = jnp.concatenate(
        [batch.astype(jnp.int32), jnp.full((NP - N,), G, jnp.int32)]
    ).reshape(NP, 1)

    degp = _sc_deg(dsts_p, starts)
    dis, h0 = _tc_prep(degp, x_p)
    agg1 = _sc_agg(h0, srcs_p, dsts_p, starts)
    h1 = _tc_layer(agg1, h0, dis, W1, b1.reshape(1, D))
    agg2 = _sc_agg(h1, srcs_p, dsts_p, starts)
    h2 = _tc_layer(agg2, h1, dis, W2, b2.reshape(1, D))
    agg3 = _sc_agg(h2, srcs_p, dsts_p, starts)
    return _tc_final(agg3, h2, dis, W3, b3.reshape(1, D), bat_p, Wl,
                     bl.reshape(1, C))


# speedup vs baseline: 5.9308x; 1.1513x over previous
import functools

import jax
import jax.numpy as jnp
from jax import lax
from jax.experimental import pallas as pl
from jax.experimental.pallas import tpu as pltpu
from jax.experimental.pallas import tpu_sc as plsc

N = 10000
NP = 10240
E = 320000
D = 128
G = 64
C = 16
CH = 128
NW = 32
RPW = NP // NW
PADN = N + 100
TCB = 1024
NT = NP // TCB
NSEG = D // 16


def _sc_mesh():
    return plsc.VectorSubcoreMesh(core_axis_name="c", subcore_axis_name="s")


@functools.partial(
    pl.kernel,
    out_type=jax.ShapeDtypeStruct((NP, 16), jnp.float32),
    mesh=_sc_mesh(),
    scratch_types=[
        pltpu.VMEM((48,), jnp.int32),
        pltpu.VMEM((CH + 16,), jnp.int32),
        pltpu.VMEM((RPW + 8, 16), jnp.float32),
    ],
)
def _sc_deg(dst_hbm, st_hbm, out_hbm, st_v, idx_d, acc):
    c = lax.axis_index("c")
    s = lax.axis_index("s")
    w = c * 16 + s
    base = w * RPW

    def _z(i, _):
        acc[i] = jnp.zeros((16,), jnp.float32)
        return 0

    lax.fori_loop(0, RPW + 8, _z, 0)

    def _zi(j, _):
        idx_d[pl.ds(j * 16, 16)] = jnp.zeros((16,), jnp.int32)
        return 0

    lax.fori_loop(0, (CH + 16) // 16, _zi, 0)

    def _zs(j, _):
        st_v[pl.ds(j * 16, 16)] = jnp.zeros((16,), jnp.int32)
        return 0

    lax.fori_loop(0, 3, _zs, 0)
    pltpu.sync_copy(st_hbm, st_v.at[pl.ds(0, 40)])
    e0 = st_v[pl.ds(w, 16)][0]
    e1 = st_v[pl.ds(w + 1, 16)][0]
    a0 = (e0 // CH) * CH
    ones = jnp.ones((16,), jnp.float32)

    def chunk(k, _):
        start = a0 + k * CH
        lo = e0 - start
        hi = jnp.minimum(e1 - start, CH)
        pltpu.sync_copy(dst_hbm.at[pl.ds(start, CH)], idx_d.at[pl.ds(0, CH)])

        def group(g, _):
            dvec = idx_d[pl.ds(g * 16, 16)]
            gbase = g * 16
            for i in range(16):
                e = gbase + i
                inb = jnp.logical_and(e >= lo, e < hi)
                r = jnp.where(inb, dvec[i] - base, jnp.int32(RPW))
                acc[r] = acc[r] + ones
            return 0

        lax.fori_loop(0, CH // 16, group, 0)
        return 0

    lax.fori_loop(0, jnp.maximum((e1 - a0 + CH - 1) // CH, 0), chunk, 0)
    pltpu.sync_copy(acc.at[pl.ds(0, RPW)], out_hbm.at[pl.ds(base, RPW)])


@functools.partial(
    pl.kernel,
    out_type=jax.ShapeDtypeStruct((NP, D), jnp.float32),
    mesh=_sc_mesh(),
    scratch_types=[
        pltpu.VMEM((48,), jnp.int32),
        pltpu.VMEM((2, CH), jnp.int32),
        pltpu.VMEM((2, CH), jnp.int32),
        pltpu.VMEM((2, CH, D), jnp.float32),
        pltpu.VMEM((RPW + 8, D), jnp.float32),
        pltpu.SemaphoreType.DMA,
    ],
)
def _sc_agg(h_hbm, src_hbm, dst_hbm, st_hbm, out_hbm, st_v, idx_s, idx_d,
            rows, acc, sem):
    c = lax.axis_index("c")
    s = lax.axis_index("s")
    w = c * 16 + s
    base = w * RPW

    def _z(i, _):
        for j in range(NSEG):
            acc[i, pl.ds(j * 16, 16)] = jnp.zeros((16,), jnp.float32)
        return 0

    lax.fori_loop(0, RPW + 8, _z, 0)

    def _zs(j, _):
        st_v[pl.ds(j * 16, 16)] = jnp.zeros((16,), jnp.int32)
        return 0

    lax.fori_loop(0, 3, _zs, 0)
    pltpu.sync_copy(st_hbm, st_v.at[pl.ds(0, 40)])
    e0 = st_v[pl.ds(w, 16)][0]
    e1 = st_v[pl.ds(w + 1, 16)][0]
    a0 = (e0 // CH) * CH
    nch = jnp.maximum((e1 - a0 + CH - 1) // CH, 0)

    def _issue(k):
        p = lax.rem(k, 2)
        start = a0 + k * CH
        pltpu.sync_copy(src_hbm.at[pl.ds(start, CH)], idx_s.at[p])
        pltpu.async_copy(h_hbm.at[idx_s.at[p]], rows.at[p], sem)

    @pl.when(nch > 0)
    def _():
        _issue(0)

    def chunk(k, carry):
        p = lax.rem(k, 2)
        start = a0 + k * CH
        lo = e0 - start
        hi = jnp.minimum(e1 - start, CH)
        pltpu.sync_copy(dst_hbm.at[pl.ds(start, CH)], idx_d.at[p])
        pltpu.make_async_copy(h_hbm.at[idx_s.at[p]], rows.at[p], sem).wait()

        @pl.when(k + 1 < nch)
        def _():
            _issue(k + 1)

        def group(g, car):
            dvec = idx_d[p, pl.ds(g * 16, 16)]
            gbase = g * 16
            for i in range(16):
                e = gbase + i
                inb = jnp.logical_and(e >= lo, e < hi)
                r = jnp.where(inb, dvec[i] - base, jnp.int32(RPW))
                for j in range(NSEG):
                    acc[r, pl.ds(j * 16, 16)] = (
                        acc[r, pl.ds(j * 16, 16)]
                        + rows[p, e, pl.ds(j * 16, 16)]
                    )
            return car

        return lax.fori_loop(0, CH // 16, group, carry)

    lax.fori_loop(0, nch, chunk, 0)
    pltpu.sync_copy(acc.at[pl.ds(0, RPW)], out_hbm.at[pl.ds(base, RPW)])


def _tc_prep_body(degp_ref, x_ref, dis_ref, h0_ref):
    t = pl.program_id(0)
    d = degp_ref[:, 0:1] + 1.0
    gid = t * TCB + lax.broadcasted_iota(jnp.int32, (TCB, 1), 0)
    dis = jnp.where(gid < N, lax.rsqrt(d), 0.0)
    dis_ref[...] = dis
    h0_ref[...] = x_ref[...] * dis


def _tc_prep(degp, x_p):
    return pl.pallas_call(
        _tc_prep_body,
        grid=(NT,),
        in_specs=[
            pl.BlockSpec((TCB, 16), lambda t: (t, 0)),
            pl.BlockSpec((TCB, D), lambda t: (t, 0)),
        ],
        out_specs=[
            pl.BlockSpec((TCB, 1), lambda t: (t, 0)),
            pl.BlockSpec((TCB, D), lambda t: (t, 0)),
        ],
        out_shape=[
            jax.ShapeDtypeStruct((NP, 1), jnp.float32),
            jax.ShapeDtypeStruct((NP, D), jnp.float32),
        ],
    )(degp, x_p)


def _tc_layer_body(acc_ref, hp_ref, dis_ref, w_ref, b_ref, out_ref):
    dis = dis_ref[...]
    u = (acc_ref[...] + hp_ref[...]) * dis
    z = jnp.dot(u, w_ref[...], preferred_element_type=jnp.float32) + b_ref[...]
    out_ref[...] = jnp.maximum(z, 0.0) * dis


def _tc_layer(agg, hp, dis, W, b):
    return pl.pallas_call(
        _tc_layer_body,
        grid=(NT,),
        in_specs=[
            pl.BlockSpec((TCB, D), lambda t: (t, 0)),
            pl.BlockSpec((TCB, D), lambda t: (t, 0)),
            pl.BlockSpec((TCB, 1), lambda t: (t, 0)),
            pl.BlockSpec((D, D), lambda t: (0, 0)),
            pl.BlockSpec((1, D), lambda t: (0, 0)),
        ],
        out_specs=pl.BlockSpec((TCB, D), lambda t: (t, 0)),
        out_shape=jax.ShapeDtypeStruct((NP, D), jnp.float32),
    )(agg, hp, dis, W, b)


def _tc_final_body(acc_ref, hp_ref, dis_ref, w_ref, b_ref, bat_ref, wl_ref,
                   bl_ref, out_ref, sums, cnts):
    t = pl.program_id(0)
    u = (acc_ref[...] + hp_ref[...]) * dis_ref[...]
    h3 = jnp.dot(u, w_ref[...], preferred_element_type=jnp.float32) + b_ref[...]
    gids = lax.broadcasted_iota(jnp.int32, (TCB, G), 1)
    P = (bat_ref[...] == gids).astype(jnp.float32)
    psum = lax.dot_general(P, h3, (((0,), (0,)), ((), ())),
                           preferred_element_type=jnp.float32)
    pcnt = lax.dot_general(P, jnp.ones((TCB, D), jnp.float32),
                           (((0,), (0,)), ((), ())),
                           preferred_element_type=jnp.float32)

    @pl.when(t == 0)
    def _():
        sums[...] = psum
        cnts[...] = pcnt

    @pl.when(t > 0)
    def _():
        sums[...] += psum
        cnts[...] += pcnt

    @pl.when(t == NT - 1)
    def _():
        means = sums[...] / jnp.maximum(cnts[...], 1.0)
        out_ref[...] = jnp.dot(means, wl_ref[...],
                               preferred_element_type=jnp.float32) + bl_ref[...]


def _tc_final(agg, hp, dis, W, b, bat, Wl, bl):
    return pl.pallas_call(
        _tc_final_body,
        grid=(NT,),
        in_specs=[
            pl.BlockSpec((TCB, D), lambda t: (t, 0)),
            pl.BlockSpec((TCB, D), lambda t: (t, 0)),
            pl.BlockSpec((TCB, 1), lambda t: (t, 0)),
            pl.BlockSpec((D, D), lambda t: (0, 0)),
            pl.BlockSpec((1, D), lambda t: (0, 0)),
            pl.BlockSpec((TCB, 1), lambda t: (t, 0)),
            pl.BlockSpec((D, C), lambda t: (0, 0)),
            pl.BlockSpec((1, C), lambda t: (0, 0)),
        ],
        out_specs=pl.BlockSpec((G, C), lambda t: (0, 0)),
        out_shape=jax.ShapeDtypeStruct((G, C), jnp.float32),
        scratch_shapes=[
            pltpu.VMEM((G, D), jnp.float32),
            pltpu.VMEM((G, D), jnp.float32),
        ],
    )(agg, hp, dis, W, b, bat, Wl, bl)


def kernel(x, edge_index, batch, W1, b1, W2, b2, W3, b3, Wl, bl):
    src = edge_index[0].astype(jnp.int32)
    dst = edge_index[1].astype(jnp.int32)
    order = jnp.argsort(dst)
    dsts = dst[order]
    srcs = src[order]
    srcs_p = jnp.concatenate([srcs, jnp.full((CH,), PADN, jnp.int32)])
    dsts_p = jnp.concatenate([dsts, jnp.full((CH,), NP - 1, jnp.int32)])
    starts = jnp.searchsorted(
        dsts, jnp.arange(NW + 1, dtype=jnp.int32) * RPW
    ).astype(jnp.int32)
    starts = jnp.concatenate([starts, jnp.zeros((7,), jnp.int32)])
    x_p = jnp.concatenate([x, jnp.zeros((NP - N, D), x.dtype)])
    bat_p = jnp.concatenate(
        [batch.astype(jnp.int32), jnp.full((NP - N,), G, jnp.int32)]
    ).reshape(NP, 1)

    degp = _sc_deg(dsts_p, starts)
    dis, h0 = _tc_prep(degp, x_p)
    agg1 = _sc_agg(h0, srcs_p, dsts_p, starts)
    h1 = _tc_layer(agg1, h0, dis, W1, b1.reshape(1, D))
    agg2 = _sc_agg(h1, srcs_p, dsts_p, starts)
    h2 = _tc_layer(agg2, h1, dis, W2, b2.reshape(1, D))
    agg3 = _sc_agg(h2, srcs_p, dsts_p, starts)
    return _tc_final(agg3, h2, dis, W3, b3.reshape(1, D), bat_p, Wl,
                     bl.reshape(1, C))
